# baseline (speedup 1.0000x reference)
import jax
import jax.numpy as jnp
import numpy as np
from jax import lax
from jax.experimental import pallas as pl
from jax.experimental.pallas import tpu as pltpu
from jax.experimental.pallas import tpu_sc as plsc

B, T, DIM = 1024, 200, 64
_NW = 32
_RPW = B // _NW
_HALF = T // 2
_NV = DIM // 16
_NB = 4

_PERM = np.concatenate([np.arange(0, 32, 2), np.arange(1, 32, 2),
                        np.arange(32, 64, 2), np.arange(33, 64, 2)])


def _fused_body(x_hbm, seg_hbm, table_hbm, segtab_hbm, pos_hbm, gamma_hbm,
                beta_hbm, out_hbm, idx_all, seg_all, rows0, rows1, rows2,
                rows3, outa, outb, pos_v, segtab_v, gb_v,
                sg0, sg1, sg2, sg3, soa, sob):
    wid = lax.axis_index("s") * 2 + lax.axis_index("c")
    b0 = wid * _RPW
    rows = [rows0, rows1, rows2, rows3]
    sem_g = [sg0, sg1, sg2, sg3]
    outs = [outa, outb]
    sem_o = [soa, sob]

    pltpu.sync_copy(pos_hbm, pos_v)
    pltpu.sync_copy(segtab_hbm, segtab_v)
    pltpu.sync_copy(gamma_hbm, gb_v.at[0])
    pltpu.sync_copy(beta_hbm, gb_v.at[1])
    pltpu.sync_copy(x_hbm.at[pl.ds(b0, _RPW)], idx_all)
    pltpu.sync_copy(seg_hbm.at[pl.ds(b0, _RPW)], seg_all.at[pl.ds(0, _RPW)])

    s0 = [segtab_v[0, pl.ds(16 * j, 16)] for j in range(_NV)]
    sd = [segtab_v[1, pl.ds(16 * j, 16)] - s0[j] for j in range(_NV)]
    gam = [gb_v[0, pl.ds(16 * j, 16)] for j in range(_NV)]
    bet = [gb_v[1, pl.ds(16 * j, 16)] for j in range(_NV)]

    @plsc.parallel_loop(0, T, unroll=2)
    def _posadd(t):
        for j in range(_NV):
            pos_v[t, pl.ds(16 * j, 16)] = pos_v[t, pl.ds(16 * j, 16)] + s0[j]

    _dn = lax.GatherDimensionNumbers(offset_dims=(), collapsed_slice_dims=(0,),
                                     start_index_map=(0,))
    _lane0 = jnp.zeros((16, 1), jnp.int32)
    _lane15 = jnp.full((16, 1), 15, jnp.int32)
    iota16 = lax.iota(jnp.int32, 16)
    cols = [2 * iota16, 2 * iota16 + 1, 2 * iota16 + 32, 2 * iota16 + 33]
    mask1 = jnp.int32(-65536)

    def _shuf(v, lane_idx):
        return lax.gather(v, lane_idx, _dn, slice_sizes=(1,),
                          mode=lax.GatherScatterMode.PROMISE_IN_BOUNDS)

    def gather_cps(r, rows_buf, sem):
        return (pltpu.make_async_copy(table_hbm.at[idx_all.at[r, 0]],
                                      rows_buf.at[pl.ds(0, _HALF)], sem),
                pltpu.make_async_copy(table_hbm.at[idx_all.at[r, 1]],
                                      rows_buf.at[pl.ds(_HALF, _HALF)], sem))

    def fire_gather(r, rows_buf, sem):
        for cp in gather_cps(r, rows_buf, sem):
            cp.start()

    def wait_gather(r, rows_buf, sem):
        for cp in gather_cps(r, rows_buf, sem):
            cp.wait()

    def compute_row(r, rows_buf, out_buf):
        @plsc.parallel_loop(0, T, unroll=4)
        def token_body(t):
            f = _shuf(seg_all[r, pl.ds(t, 16)].astype(jnp.float32), _lane0)
            w = [plsc.bitcast(rows_buf[t, pl.ds(32 * h, 32)], jnp.int32)
                 for h in range(2)]
            tok = []
            for h in range(2):
                tok.append(plsc.bitcast(w[h] << 16, jnp.float32))
                tok.append(plsc.bitcast(w[h] & mask1, jnp.float32))
            e = [tok[j] + pos_v[t, pl.ds(16 * j, 16)] + f * sd[j]
                 for j in range(_NV)]
            tot = _shuf(plsc.cumsum(e[0] + e[1] + e[2] + e[3]), _lane15)
            totq = _shuf(plsc.cumsum(e[0] * e[0] + e[1] * e[1]
                                     + e[2] * e[2] + e[3] * e[3]), _lane15)
            mean = tot * (1.0 / DIM)
            v16 = totq * (1.0 / DIM) - mean * mean + 1e-5
            seed = plsc.bitcast(
                jnp.int32(0x5F3759DF) - (plsc.bitcast(v16, jnp.int32) >> 1),
                jnp.float32)
            half = v16 * 0.5
            r0 = seed * (1.5 - half * seed * seed)
            rs = r0 * (1.5 - half * r0 * r0)
            mrs = mean * rs
            for j in range(_NV):
                gs = gam[j] * rs
                y = e[j] * gs + (bet[j] - mrs * gam[j])
                plsc.store_scatter(out_buf, [jnp.full((16,), t, jnp.int32),
                                             cols[j]], y)

    for k in range(_NB):
        fire_gather(k, rows[k], sem_g[k])

    def quad_body(i, _):
        for k in range(_NB):
            r = _NB * i + k
            o = k % 2
            wait_gather(r, rows[k], sem_g[k])
            if k >= 2:
                pltpu.make_async_copy(outs[o], out_hbm.at[b0 + r - 2],
                                      sem_o[o]).wait()
            else:
                @pl.when(i > 0)
                def _():
                    pltpu.make_async_copy(outs[o], out_hbm.at[b0 + r - 2],
                                          sem_o[o]).wait()
            compute_row(r, rows[k], outs[o])
            pltpu.async_copy(outs[o], out_hbm.at[b0 + r], sem_o[o])

            @pl.when(r + _NB < _RPW)
            def _():
                fire_gather(r + _NB, rows[k], sem_g[k])
        return ()

    lax.fori_loop(0, _RPW // _NB, quad_body, ())

    pltpu.make_async_copy(outs[0], out_hbm.at[b0 + _RPW - 2], sem_o[0]).wait()
    pltpu.make_async_copy(outs[1], out_hbm.at[b0 + _RPW - 1], sem_o[1]).wait()


def kernel(x, segment, tok_table, seg_table, pos_table, gamma, beta):
    x3 = x.astype(jnp.int32).reshape(B, 2, _HALF)
    seg = segment.astype(jnp.int32)
    tok16 = tok_table.astype(jnp.bfloat16)
    perm = jnp.asarray(_PERM)
    segtab_p = seg_table[:, perm]
    pos_p = pos_table[:T][:, perm]
    gamma_p = gamma[perm]
    beta_p = beta[perm]
    mesh = plsc.VectorSubcoreMesh(core_axis_name="c", subcore_axis_name="s")
    fused = pl.kernel(
        _fused_body,
        out_type=jax.ShapeDtypeStruct((B, T, DIM), jnp.float32),
        mesh=mesh,
        scratch_types=[
            pltpu.VMEM((_RPW, 2, _HALF), jnp.int32),
            pltpu.VMEM((_RPW + 1, T), jnp.int32),
            pltpu.VMEM((T, DIM), jnp.bfloat16),
            pltpu.VMEM((T, DIM), jnp.bfloat16),
            pltpu.VMEM((T, DIM), jnp.bfloat16),
            pltpu.VMEM((T, DIM), jnp.bfloat16),
            pltpu.VMEM((T, DIM), jnp.float32),
            pltpu.VMEM((T, DIM), jnp.float32),
            pltpu.VMEM((T, DIM), jnp.float32),
            pltpu.VMEM((2, DIM), jnp.float32),
            pltpu.VMEM((2, DIM), jnp.float32),
            pltpu.SemaphoreType.DMA,
            pltpu.SemaphoreType.DMA,
            pltpu.SemaphoreType.DMA,
            pltpu.SemaphoreType.DMA,
            pltpu.SemaphoreType.DMA,
            pltpu.SemaphoreType.DMA,
        ],
        compiler_params=pltpu.CompilerParams(use_tc_tiling_on_sc=False,
                                             needs_layout_passes=False),
    )
    return fused(x3, seg, tok16, segtab_p, pos_p, gamma_p, beta_p)


# speedup vs baseline: 1.1612x; 1.1612x over previous
import jax
import jax.numpy as jnp
from jax import lax
from jax.experimental import pallas as pl
from jax.experimental.pallas import tpu as pltpu
from jax.experimental.pallas import tpu_sc as plsc

B, T, DIM = 1024, 200, 64
_NW = 32
_RPW = B // _NW
_HALF = T // 2
_NV = DIM // 16
_NB = 4


def _fused_body(x_hbm, seg_hbm, table_hbm, segtab_hbm, pos_hbm, gamma_hbm,
                beta_hbm, out_hbm, idx_all, seg_all, rows0, rows1, rows2,
                rows3, outa, outb, pos_v, segtab_v, gb_v,
                sg0, sg1, sg2, sg3, soa, sob):
    wid = lax.axis_index("s") * 2 + lax.axis_index("c")
    b0 = wid * _RPW
    rows = [rows0, rows1, rows2, rows3]
    sem_g = [sg0, sg1, sg2, sg3]
    outs = [outa, outb]
    sem_o = [soa, sob]

    pltpu.sync_copy(pos_hbm.at[pl.ds(0, T)], pos_v)
    pltpu.sync_copy(segtab_hbm, segtab_v)
    pltpu.sync_copy(gamma_hbm, gb_v.at[0])
    pltpu.sync_copy(beta_hbm, gb_v.at[1])
    pltpu.sync_copy(x_hbm.at[pl.ds(b0, _RPW)], idx_all)
    pltpu.sync_copy(seg_hbm.at[pl.ds(b0, _RPW)], seg_all.at[pl.ds(0, _RPW)])

    s0 = [segtab_v[0, pl.ds(16 * j, 16)] for j in range(_NV)]
    sd = [segtab_v[1, pl.ds(16 * j, 16)] - s0[j] for j in range(_NV)]
    gam = [gb_v[0, pl.ds(16 * j, 16)] for j in range(_NV)]
    bet = [gb_v[1, pl.ds(16 * j, 16)] for j in range(_NV)]

    @plsc.parallel_loop(0, T, unroll=2)
    def _posadd(t):
        for j in range(_NV):
            pos_v[t, pl.ds(16 * j, 16)] = pos_v[t, pl.ds(16 * j, 16)] + s0[j]

    _dn = lax.GatherDimensionNumbers(offset_dims=(), collapsed_slice_dims=(0,),
                                     start_index_map=(0,))
    _lane0 = jnp.zeros((16, 1), jnp.int32)
    _lane15 = jnp.full((16, 1), 15, jnp.int32)

    def _shuf(v, lane_idx):
        return lax.gather(v, lane_idx, _dn, slice_sizes=(1,),
                          mode=lax.GatherScatterMode.PROMISE_IN_BOUNDS)

    def gather_cps(r, rows_buf, sem):
        return (pltpu.make_async_copy(table_hbm.at[idx_all.at[r, pl.ds(0, 96)]],
                                      rows_buf.at[pl.ds(0, 96)], sem),
                pltpu.make_async_copy(table_hbm.at[idx_all.at[r, pl.ds(96, 104)]],
                                      rows_buf.at[pl.ds(96, 104)], sem))

    def fire_gather(r, rows_buf, sem):
        for cp in gather_cps(r, rows_buf, sem):
            cp.start()

    def wait_gather(r, rows_buf, sem):
        for cp in gather_cps(r, rows_buf, sem):
            cp.wait()

    def compute_row(r, rows_buf, out_buf):
        @plsc.parallel_loop(0, T, unroll=4)
        def token_body(t):
            f = _shuf(seg_all[r, pl.ds(t, 16)].astype(jnp.float32), _lane0)
            e = [rows_buf[t, pl.ds(16 * j, 16)] + pos_v[t, pl.ds(16 * j, 16)]
                 + f * sd[j] for j in range(_NV)]
            tot = _shuf(plsc.cumsum(e[0] + e[1] + e[2] + e[3]), _lane15)
            totq = _shuf(plsc.cumsum(e[0] * e[0] + e[1] * e[1]
                                     + e[2] * e[2] + e[3] * e[3]), _lane15)
            mean = tot * (1.0 / DIM)
            v16 = totq * (1.0 / DIM) - mean * mean + 1e-5
            seed = plsc.bitcast(
                jnp.int32(0x5F3759DF) - (plsc.bitcast(v16, jnp.int32) >> 1),
                jnp.float32)
            half = v16 * 0.5
            r0 = seed * (1.5 - half * seed * seed)
            rs = r0 * (1.5 - half * r0 * r0)
            mrs = mean * rs
            for j in range(_NV):
                gs = gam[j] * rs
                out_buf[t, pl.ds(16 * j, 16)] = e[j] * gs + (bet[j] - mrs * gam[j])

    for k in range(_NB):
        fire_gather(k, rows[k], sem_g[k])

    def quad_body(i, _):
        for k in range(_NB):
            r = _NB * i + k
            o = k % 2
            wait_gather(r, rows[k], sem_g[k])
            if k >= 2:
                pltpu.make_async_copy(outs[o], out_hbm.at[b0 + r - 2],
                                      sem_o[o]).wait()
            else:
                @pl.when(i > 0)
                def _():
                    pltpu.make_async_copy(outs[o], out_hbm.at[b0 + r - 2],
                                          sem_o[o]).wait()
            compute_row(r, rows[k], outs[o])
            pltpu.async_copy(outs[o], out_hbm.at[b0 + r], sem_o[o])

            @pl.when(r + _NB < _RPW)
            def _():
                fire_gather(r + _NB, rows[k], sem_g[k])
        return ()

    lax.fori_loop(0, _RPW // _NB, quad_body, ())

    pltpu.make_async_copy(outs[0], out_hbm.at[b0 + _RPW - 2], sem_o[0]).wait()
    pltpu.make_async_copy(outs[1], out_hbm.at[b0 + _RPW - 1], sem_o[1]).wait()


def kernel(x, segment, tok_table, seg_table, pos_table, gamma, beta):
    xi = x.astype(jnp.int32)
    seg = segment.astype(jnp.int32)
    mesh = plsc.VectorSubcoreMesh(core_axis_name="c", subcore_axis_name="s")
    fused = pl.kernel(
        _fused_body,
        out_type=jax.ShapeDtypeStruct((B, T, DIM), jnp.float32),
        mesh=mesh,
        scratch_types=[
            pltpu.VMEM((_RPW, T), jnp.int32),
            pltpu.VMEM((_RPW + 1, T), jnp.int32),
            pltpu.VMEM((T, DIM), jnp.float32),
            pltpu.VMEM((T, DIM), jnp.float32),
            pltpu.VMEM((T, DIM), jnp.float32),
            pltpu.VMEM((T, DIM), jnp.float32),
            pltpu.VMEM((T, DIM), jnp.float32),
            pltpu.VMEM((T, DIM), jnp.float32),
            pltpu.VMEM((T, DIM), jnp.float32),
            pltpu.VMEM((2, DIM), jnp.float32),
            pltpu.VMEM((2, DIM), jnp.float32),
            pltpu.SemaphoreType.DMA,
            pltpu.SemaphoreType.DMA,
            pltpu.SemaphoreType.DMA,
            pltpu.SemaphoreType.DMA,
            pltpu.SemaphoreType.DMA,
            pltpu.SemaphoreType.DMA,
        ],
        compiler_params=pltpu.CompilerParams(use_tc_tiling_on_sc=False,
                                             needs_layout_passes=False),
    )
    return fused(xi, seg, tok_table, seg_table, pos_table, gamma, beta)


# speedup vs baseline: 1.1757x; 1.0125x over previous
import jax
import jax.numpy as jnp
from jax import lax
from jax.experimental import pallas as pl
from jax.experimental.pallas import tpu as pltpu
from jax.experimental.pallas import tpu_sc as plsc

B, T, DIM = 1024, 200, 64
_NW = 32
_RPW = B // _NW
_HALF = T // 2
_NV = DIM // 16
_NB = 4


def _fused_body(x_hbm, table_hbm, segtab_hbm, pos_hbm, gamma_hbm,
                beta_hbm, out_hbm, idx_all, seg_all, rows0, rows1, rows2,
                rows3, outa, outb, pos_v, segtab_v, gb_v,
                sg0, sg1, sg2, sg3, soa, sob):
    wid = lax.axis_index("s") * 2 + lax.axis_index("c")
    b0 = wid * _RPW
    rows = [rows0, rows1, rows2, rows3]
    sem_g = [sg0, sg1, sg2, sg3]
    outs = [outa, outb]
    sem_o = [soa, sob]

    pltpu.sync_copy(pos_hbm.at[pl.ds(0, T)], pos_v)
    pltpu.sync_copy(segtab_hbm, segtab_v)
    pltpu.sync_copy(gamma_hbm, gb_v.at[0])
    pltpu.sync_copy(beta_hbm, gb_v.at[1])
    pltpu.sync_copy(x_hbm.at[pl.ds(b0, _RPW)], idx_all.at[:, pl.ds(0, T)])

    def unpack_row(r, _):
        @plsc.parallel_loop(0, 208, 16)
        def _unpack(c):
            w = idx_all[r, pl.ds(c, 16)]
            idx_all[r, pl.ds(c, 16)] = w & 131071
            seg_all[r, pl.ds(c, 16)] = (w >> 17).astype(jnp.float32)
        return ()

    lax.fori_loop(0, _RPW, unpack_row, ())

    s0 = [segtab_v[0, pl.ds(16 * j, 16)] for j in range(_NV)]
    sd = [segtab_v[1, pl.ds(16 * j, 16)] - s0[j] for j in range(_NV)]
    gam = [gb_v[0, pl.ds(16 * j, 16)] for j in range(_NV)]
    bet = [gb_v[1, pl.ds(16 * j, 16)] for j in range(_NV)]

    @plsc.parallel_loop(0, T, unroll=2)
    def _posadd(t):
        for j in range(_NV):
            pos_v[t, pl.ds(16 * j, 16)] = pos_v[t, pl.ds(16 * j, 16)] + s0[j]

    _dn = lax.GatherDimensionNumbers(offset_dims=(), collapsed_slice_dims=(0,),
                                     start_index_map=(0,))
    _lane0 = jnp.zeros((16, 1), jnp.int32)
    _lane15 = jnp.full((16, 1), 15, jnp.int32)

    def _shuf(v, lane_idx):
        return lax.gather(v, lane_idx, _dn, slice_sizes=(1,),
                          mode=lax.GatherScatterMode.PROMISE_IN_BOUNDS)

    def gather_cps(r, rows_buf, sem):
        return (pltpu.make_async_copy(table_hbm.at[idx_all.at[r, pl.ds(0, 96)]],
                                      rows_buf.at[pl.ds(0, 96)], sem),
                pltpu.make_async_copy(table_hbm.at[idx_all.at[r, pl.ds(96, 104)]],
                                      rows_buf.at[pl.ds(96, 104)], sem))

    def fire_gather(r, rows_buf, sem):
        for cp in gather_cps(r, rows_buf, sem):
            cp.start()

    def wait_gather(r, rows_buf, sem):
        for cp in gather_cps(r, rows_buf, sem):
            cp.wait()

    def compute_row(r, rows_buf, out_buf):
        @plsc.parallel_loop(0, T, unroll=4)
        def token_body(t):
            f = _shuf(seg_all[r, pl.ds(t, 16)], _lane0)
            e = [rows_buf[t, pl.ds(16 * j, 16)] + pos_v[t, pl.ds(16 * j, 16)]
                 + f * sd[j] for j in range(_NV)]
            tot = _shuf(plsc.cumsum(e[0] + e[1] + e[2] + e[3]), _lane15)
            totq = _shuf(plsc.cumsum(e[0] * e[0] + e[1] * e[1]
                                     + e[2] * e[2] + e[3] * e[3]), _lane15)
            mean = tot * (1.0 / DIM)
            v16 = totq * (1.0 / DIM) - mean * mean + 1e-5
            seed = plsc.bitcast(
                jnp.int32(0x5F3759DF) - (plsc.bitcast(v16, jnp.int32) >> 1),
                jnp.float32)
            half = v16 * 0.5
            r0 = seed * (1.5 - half * seed * seed)
            rs = r0 * (1.5 - half * r0 * r0)
            mrs = mean * rs
            for j in range(_NV):
                gs = gam[j] * rs
                out_buf[t, pl.ds(16 * j, 16)] = e[j] * gs + (bet[j] - mrs * gam[j])

    for k in range(_NB):
        fire_gather(k, rows[k], sem_g[k])

    def quad_body(i, _):
        for k in range(_NB):
            r = _NB * i + k
            o = k % 2
            wait_gather(r, rows[k], sem_g[k])
            if k >= 2:
                pltpu.make_async_copy(outs[o], out_hbm.at[b0 + r - 2],
                                      sem_o[o]).wait()
            else:
                @pl.when(i > 0)
                def _():
                    pltpu.make_async_copy(outs[o], out_hbm.at[b0 + r - 2],
                                          sem_o[o]).wait()
            compute_row(r, rows[k], outs[o])
            pltpu.async_copy(outs[o], out_hbm.at[b0 + r], sem_o[o])

            @pl.when(r + _NB < _RPW)
            def _():
                fire_gather(r + _NB, rows[k], sem_g[k])
        return ()

    lax.fori_loop(0, _RPW // _NB, quad_body, ())

    pltpu.make_async_copy(outs[0], out_hbm.at[b0 + _RPW - 2], sem_o[0]).wait()
    pltpu.make_async_copy(outs[1], out_hbm.at[b0 + _RPW - 1], sem_o[1]).wait()


def kernel(x, segment, tok_table, seg_table, pos_table, gamma, beta):
    packed = x.astype(jnp.int32) | (segment.astype(jnp.int32) << 17)
    mesh = plsc.VectorSubcoreMesh(core_axis_name="c", subcore_axis_name="s")
    fused = pl.kernel(
        _fused_body,
        out_type=jax.ShapeDtypeStruct((B, T, DIM), jnp.float32),
        mesh=mesh,
        scratch_types=[
            pltpu.VMEM((_RPW, 216), jnp.int32),
            pltpu.VMEM((_RPW, 216), jnp.float32),
            pltpu.VMEM((T, DIM), jnp.float32),
            pltpu.VMEM((T, DIM), jnp.float32),
            pltpu.VMEM((T, DIM), jnp.float32),
            pltpu.VMEM((T, DIM), jnp.float32),
            pltpu.VMEM((T, DIM), jnp.float32),
            pltpu.VMEM((T, DIM), jnp.float32),
            pltpu.VMEM((T, DIM), jnp.float32),
            pltpu.VMEM((2, DIM), jnp.float32),
            pltpu.VMEM((2, DIM), jnp.float32),
            pltpu.SemaphoreType.DMA,
            pltpu.SemaphoreType.DMA,
            pltpu.SemaphoreType.DMA,
            pltpu.SemaphoreType.DMA,
            pltpu.SemaphoreType.DMA,
            pltpu.SemaphoreType.DMA,
        ],
        compiler_params=pltpu.CompilerParams(use_tc_tiling_on_sc=False,
                                             needs_layout_passes=False),
    )
    return fused(packed, tok_table, seg_table, pos_table, gamma, beta)
